# baseline (speedup 1.0000x reference)
import functools

import jax
import jax.numpy as jnp
from jax.experimental import pallas as pl
from jax.experimental.pallas import tpu as pltpu

B, N, C, H, G, KD = 4, 512, 512, 8, 3, 16
D = C // H
GRP = 16
NG = N // GRP


def _swish(x):
    return x * (0.5 * jnp.tanh(0.5 * x) + 0.5)


def _fused_kernel(g_ref, x_ref, ln1_g_ref, ln1_b_ref, wqkv_ref, bqkv_ref,
                  w0p_ref, b0p_ref, w1p_ref, b1p_ref, w2p_ref, b2p_ref,
                  wo_ref, bo_ref, ln2_g_ref, ln2_b_ref,
                  wm1_ref, bm1_ref, wm2_ref, bm2_ref,
                  y_ref):
    f32 = jnp.float32
    x = x_ref[0]

    mu = jnp.mean(x, axis=-1, keepdims=True)
    var = jnp.mean(jnp.square(x), axis=-1, keepdims=True) - jnp.square(mu)
    hq = (x - mu) * jax.lax.rsqrt(var + 1e-5) * ln1_g_ref[...] + ln1_b_ref[...]
    qkv = jnp.dot(hq, wqkv_ref[...], preferred_element_type=f32) + bqkv_ref[...]
    q = qkv[:, :C]
    kt = qkv[:, C:2 * C].T
    v = qkv[:, 2 * C:]

    bf16 = jnp.bfloat16
    g = g_ref[0]
    l2s = []
    for mc in range(NG):
        gg = g[:, mc * G * GRP:(mc + 1) * G * GRP]
        l1 = _swish(jnp.dot(gg, w0p_ref[...], preferred_element_type=f32)
                    + b0p_ref[...])
        l2 = _swish(jnp.dot(l1.astype(bf16), w1p_ref[...],
                            preferred_element_type=f32) + b1p_ref[...])
        l2s.append(l2.astype(bf16))
    l3s = []
    for t in range(NG // 2):
        pair = jnp.concatenate([l2s[2 * t], l2s[2 * t + 1]], axis=1)
        l3s.append(jnp.dot(pair, w2p_ref[...], preferred_element_type=f32)
                   + b2p_ref[...])

    outs = []
    for hh in range(H):
        loc = jnp.concatenate(
            [l3s[t][:, hh * 2 * GRP:(hh + 1) * 2 * GRP]
             for t in range(NG // 2)], axis=1)
        qh = q[:, hh * D:(hh + 1) * D]
        kth = kt[hh * D:(hh + 1) * D, :]
        s = jnp.dot(qh, kth, preferred_element_type=f32) * (1.0 / 8.0) + loc
        mx = jnp.max(s, axis=1, keepdims=True)
        e = jnp.exp(s - mx)
        p = e / jnp.sum(e, axis=1, keepdims=True)
        vh = v[:, hh * D:(hh + 1) * D]
        outs.append(jnp.dot(p, vh, preferred_element_type=f32))
    att = jnp.concatenate(outs, axis=1)

    o = jnp.dot(att, wo_ref[...], preferred_element_type=f32) + bo_ref[...]
    x1 = x + o

    mu2 = jnp.mean(x1, axis=-1, keepdims=True)
    var2 = jnp.mean(jnp.square(x1), axis=-1, keepdims=True) - jnp.square(mu2)
    hn = (x1 - mu2) * jax.lax.rsqrt(var2 + 1e-5) * ln2_g_ref[...] \
        + ln2_b_ref[...]
    mid = _swish(jnp.dot(hn, wm1_ref[...], preferred_element_type=f32)
                 + bm1_ref[...])
    y = x1 + jnp.dot(mid, wm2_ref[...], preferred_element_type=f32) \
        + bm2_ref[...]
    y_ref[0] = y


@functools.partial(jax.jit, static_argnums=())
def kernel(pairwise_g, coset_functions, mask, ln1_g, ln1_b,
           wn_w0, wn_b0, wn_w1, wn_b1, wn_w2, wn_b2,
           Wq, bq, Wk, bk, Wi, bi, Wo, bo,
           ln2_g, ln2_b, Wm1, bm1, Wm2, bm2):
    f32 = jnp.float32
    x = coset_functions
    wqkv = jnp.concatenate([Wq, Wk, Wi], axis=1)
    bqkv = jnp.concatenate([bq, bk, bi]).reshape(1, 3 * C)

    bf16 = jnp.bfloat16
    eye = jnp.eye(GRP, dtype=f32)
    w0p = jnp.kron(eye, wn_w0).astype(bf16)
    b0p = jnp.tile(wn_b0, GRP).reshape(1, KD * GRP)
    w1p = jnp.kron(eye, wn_w1).astype(bf16)
    b1p = jnp.tile(wn_b1, GRP).reshape(1, KD * GRP)
    G3 = 2 * GRP
    w2s = jnp.kron(jnp.eye(G3, dtype=f32), wn_w2)
    perm = [(t % G3) * H + t // G3 for t in range(H * G3)]
    w2p = w2s[:, jnp.array(perm)].astype(bf16)
    b2p = jnp.repeat(wn_b2, G3).reshape(1, H * G3)

    g2 = pairwise_g.reshape(B, N, N * G).astype(bf16)

    call = pl.pallas_call(
        _fused_kernel,
        grid=(1,),
        in_specs=[
            pl.BlockSpec((1, N, N * G), lambda b: (0, 0, 0)),
            pl.BlockSpec((1, N, C), lambda b: (0, 0, 0)),
            pl.BlockSpec((1, C), lambda b: (0, 0)),
            pl.BlockSpec((1, C), lambda b: (0, 0)),
            pl.BlockSpec((C, 3 * C), lambda b: (0, 0)),
            pl.BlockSpec((1, 3 * C), lambda b: (0, 0)),
            pl.BlockSpec((G * GRP, KD * GRP), lambda b: (0, 0)),
            pl.BlockSpec((1, KD * GRP), lambda b: (0, 0)),
            pl.BlockSpec((KD * GRP, KD * GRP), lambda b: (0, 0)),
            pl.BlockSpec((1, KD * GRP), lambda b: (0, 0)),
            pl.BlockSpec((KD * 2 * GRP, H * 2 * GRP), lambda b: (0, 0)),
            pl.BlockSpec((1, H * 2 * GRP), lambda b: (0, 0)),
            pl.BlockSpec((C, C), lambda b: (0, 0)),
            pl.BlockSpec((1, C), lambda b: (0, 0)),
            pl.BlockSpec((1, C), lambda b: (0, 0)),
            pl.BlockSpec((1, C), lambda b: (0, 0)),
            pl.BlockSpec((C, C), lambda b: (0, 0)),
            pl.BlockSpec((1, C), lambda b: (0, 0)),
            pl.BlockSpec((C, C), lambda b: (0, 0)),
            pl.BlockSpec((1, C), lambda b: (0, 0)),
        ],
        out_specs=pl.BlockSpec((1, N, C), lambda b: (0, 0, 0)),
        out_shape=jax.ShapeDtypeStruct((1, N, C), f32),
    )
    ys = [call(g2[b:b + 1], x[b:b + 1],
               ln1_g.reshape(1, C), ln1_b.reshape(1, C), wqkv, bqkv,
               w0p, b0p, w1p, b1p, w2p, b2p,
               Wo, bo.reshape(1, C), ln2_g.reshape(1, C),
               ln2_b.reshape(1, C),
               Wm1, bm1.reshape(1, C), Wm2, bm2.reshape(1, C))
          for b in range(B)]
    y = jnp.concatenate(ys, axis=0)

    return (pairwise_g, y, mask)


# speedup vs baseline: 1.4057x; 1.4057x over previous
import functools

import jax
import jax.numpy as jnp
from jax.experimental import pallas as pl
from jax.experimental.pallas import tpu as pltpu

B, N, C, H, G, KD = 4, 512, 512, 8, 3, 16
D = C // H
GRP = 16
NG = N // GRP


def _swish(x):
    return x * (0.5 * jnp.tanh(0.5 * x) + 0.5)


def _fused_kernel(g_ref, x_ref, ln1_g_ref, ln1_b_ref, wqkv_ref, bqkv_ref,
                  w0p_ref, b0p_ref, w1p_ref, b1p_ref, w2p_ref, b2p_ref,
                  wo_ref, bo_ref, ln2_g_ref, ln2_b_ref,
                  wm1_ref, bm1_ref, wm2_ref, bm2_ref,
                  y_ref):
    f32 = jnp.float32
    x = x_ref[0]

    mu = jnp.mean(x, axis=-1, keepdims=True)
    var = jnp.mean(jnp.square(x), axis=-1, keepdims=True) - jnp.square(mu)
    hq = (x - mu) * jax.lax.rsqrt(var + 1e-5) * ln1_g_ref[...] + ln1_b_ref[...]
    qkv = jnp.dot(hq, wqkv_ref[...], preferred_element_type=f32) + bqkv_ref[...]
    q = qkv[:, :C]
    kt = qkv[:, C:2 * C].T
    v = qkv[:, 2 * C:]

    bf16 = jnp.bfloat16
    g = g_ref[0]
    l2s = []
    for mc in range(NG):
        gg = g[:, mc * G * GRP:(mc + 1) * G * GRP]
        l1 = _swish(jnp.dot(gg, w0p_ref[...], preferred_element_type=f32)
                    + b0p_ref[...])
        l2 = _swish(jnp.dot(l1.astype(bf16), w1p_ref[...],
                            preferred_element_type=f32) + b1p_ref[...])
        l2s.append(l2.astype(bf16))
    l3s = []
    for t in range(NG // 2):
        pair = jnp.concatenate([l2s[2 * t], l2s[2 * t + 1]], axis=1)
        l3s.append(jnp.dot(pair, w2p_ref[...], preferred_element_type=f32)
                   + b2p_ref[...])

    outs = []
    for hh in range(H):
        loc = jnp.concatenate(
            [l3s[t][:, hh * 2 * GRP:(hh + 1) * 2 * GRP]
             for t in range(NG // 2)], axis=1)
        qh = q[:, hh * D:(hh + 1) * D]
        kth = kt[hh * D:(hh + 1) * D, :]
        s = jnp.dot(qh, kth, preferred_element_type=f32) * (1.0 / 8.0) + loc
        mx = jnp.max(s, axis=1, keepdims=True)
        e = jnp.exp(s - mx)
        p = e / jnp.sum(e, axis=1, keepdims=True)
        vh = v[:, hh * D:(hh + 1) * D]
        outs.append(jnp.dot(p, vh, preferred_element_type=f32))
    att = jnp.concatenate(outs, axis=1)

    o = jnp.dot(att, wo_ref[...], preferred_element_type=f32) + bo_ref[...]
    x1 = x + o

    mu2 = jnp.mean(x1, axis=-1, keepdims=True)
    var2 = jnp.mean(jnp.square(x1), axis=-1, keepdims=True) - jnp.square(mu2)
    hn = (x1 - mu2) * jax.lax.rsqrt(var2 + 1e-5) * ln2_g_ref[...] \
        + ln2_b_ref[...]
    mid = _swish(jnp.dot(hn, wm1_ref[...], preferred_element_type=f32)
                 + bm1_ref[...])
    y = x1 + jnp.dot(mid, wm2_ref[...], preferred_element_type=f32) \
        + bm2_ref[...]
    y_ref[0] = y


@functools.partial(jax.jit, static_argnums=())
def kernel(pairwise_g, coset_functions, mask, ln1_g, ln1_b,
           wn_w0, wn_b0, wn_w1, wn_b1, wn_w2, wn_b2,
           Wq, bq, Wk, bk, Wi, bi, Wo, bo,
           ln2_g, ln2_b, Wm1, bm1, Wm2, bm2):
    f32 = jnp.float32
    x = coset_functions
    wqkv = jnp.concatenate([Wq, Wk, Wi], axis=1)
    bqkv = jnp.concatenate([bq, bk, bi]).reshape(1, 3 * C)

    bf16 = jnp.bfloat16
    eye = jnp.eye(GRP, dtype=f32)
    w0p = jnp.kron(eye, wn_w0).astype(bf16)
    b0p = jnp.tile(wn_b0, GRP).reshape(1, KD * GRP)
    w1p = jnp.kron(eye, wn_w1).astype(bf16)
    b1p = jnp.tile(wn_b1, GRP).reshape(1, KD * GRP)
    G3 = 2 * GRP
    w2s = jnp.kron(jnp.eye(G3, dtype=f32), wn_w2)
    perm = [(t % G3) * H + t // G3 for t in range(H * G3)]
    w2p = w2s[:, jnp.array(perm)].astype(bf16)
    b2p = jnp.repeat(wn_b2, G3).reshape(1, H * G3)

    g2 = jnp.where(mask[:, :, None], pairwise_g.reshape(B, N, N * G),
                   jnp.float32(0)).astype(bf16)

    call = pl.pallas_call(
        _fused_kernel,
        grid=(B,),
        in_specs=[
            pl.BlockSpec((1, N, N * G), lambda b: (b, 0, 0)),
            pl.BlockSpec((1, N, C), lambda b: (b, 0, 0)),
            pl.BlockSpec((1, C), lambda b: (0, 0)),
            pl.BlockSpec((1, C), lambda b: (0, 0)),
            pl.BlockSpec((C, 3 * C), lambda b: (0, 0)),
            pl.BlockSpec((1, 3 * C), lambda b: (0, 0)),
            pl.BlockSpec((G * GRP, KD * GRP), lambda b: (0, 0)),
            pl.BlockSpec((1, KD * GRP), lambda b: (0, 0)),
            pl.BlockSpec((KD * GRP, KD * GRP), lambda b: (0, 0)),
            pl.BlockSpec((1, KD * GRP), lambda b: (0, 0)),
            pl.BlockSpec((KD * 2 * GRP, H * 2 * GRP), lambda b: (0, 0)),
            pl.BlockSpec((1, H * 2 * GRP), lambda b: (0, 0)),
            pl.BlockSpec((C, C), lambda b: (0, 0)),
            pl.BlockSpec((1, C), lambda b: (0, 0)),
            pl.BlockSpec((1, C), lambda b: (0, 0)),
            pl.BlockSpec((1, C), lambda b: (0, 0)),
            pl.BlockSpec((C, C), lambda b: (0, 0)),
            pl.BlockSpec((1, C), lambda b: (0, 0)),
            pl.BlockSpec((C, C), lambda b: (0, 0)),
            pl.BlockSpec((1, C), lambda b: (0, 0)),
        ],
        out_specs=pl.BlockSpec((1, N, C), lambda b: (b, 0, 0)),
        out_shape=jax.ShapeDtypeStruct((B, N, C), f32),
    )
    y = call(g2, x, ln1_g.reshape(1, C), ln1_b.reshape(1, C), wqkv, bqkv,
             w0p, b0p, w1p, b1p, w2p, b2p,
             Wo, bo.reshape(1, C), ln2_g.reshape(1, C), ln2_b.reshape(1, C),
             Wm1, bm1.reshape(1, C), Wm2, bm2.reshape(1, C))

    return (pairwise_g, y, mask)


# speedup vs baseline: 1.4198x; 1.0101x over previous
import functools

import jax
import jax.numpy as jnp
from jax.experimental import pallas as pl
from jax.experimental.pallas import tpu as pltpu

B, N, C, H, G, KD = 4, 512, 512, 8, 3, 16
D = C // H
GRP = 16
NG = N // GRP


def _swish(x):
    return x * (0.5 * jnp.tanh(0.5 * x) + 0.5)


def _fused_kernel(g_ref, x_ref, ln1_g_ref, ln1_b_ref, wqkv_ref, bqkv_ref,
                  w0p_ref, b0p_ref, w1p_ref, b1p_ref, w2p_ref, b2p_ref,
                  wo_ref, bo_ref, ln2_g_ref, ln2_b_ref,
                  wm1_ref, bm1_ref, wm2_ref, bm2_ref,
                  y_ref):
    f32 = jnp.float32
    x = x_ref[0]

    mu = jnp.mean(x, axis=-1, keepdims=True)
    var = jnp.mean(jnp.square(x), axis=-1, keepdims=True) - jnp.square(mu)
    hq = (x - mu) * jax.lax.rsqrt(var + 1e-5) * ln1_g_ref[...] + ln1_b_ref[...]
    qkv = jnp.dot(hq, wqkv_ref[...], preferred_element_type=f32) + bqkv_ref[...]
    q = qkv[:, :C]
    kt = qkv[:, C:2 * C].T
    v = qkv[:, 2 * C:]

    bf16 = jnp.bfloat16
    g = g_ref[0]
    l2s = []
    for mc in range(NG):
        gg = g[:, mc * G * GRP:(mc + 1) * G * GRP]
        l1 = _swish(jnp.dot(gg, w0p_ref[...], preferred_element_type=f32)
                    + b0p_ref[...])
        l2 = _swish(jnp.dot(l1.astype(bf16), w1p_ref[...],
                            preferred_element_type=f32) + b1p_ref[...])
        l2s.append(l2.astype(bf16))
    l3s = []
    for t in range(NG // 2):
        pair = jnp.concatenate([l2s[2 * t], l2s[2 * t + 1]], axis=1)
        l3s.append(jnp.dot(pair, w2p_ref[...], preferred_element_type=f32)
                   + b2p_ref[...])

    outs = []
    for hh in range(H):
        loc = jnp.concatenate(
            [l3s[t][:, hh * 2 * GRP:(hh + 1) * 2 * GRP]
             for t in range(NG // 2)], axis=1)
        qh = q[:, hh * D:(hh + 1) * D]
        kth = kt[hh * D:(hh + 1) * D, :]
        s = jnp.dot(qh, kth, preferred_element_type=f32) * (1.0 / 8.0) + loc
        mx = jnp.max(s, axis=1, keepdims=True)
        e = jnp.exp(s - mx)
        p = e / jnp.sum(e, axis=1, keepdims=True)
        vh = v[:, hh * D:(hh + 1) * D]
        outs.append(jnp.dot(p, vh, preferred_element_type=f32))
    att = jnp.concatenate(outs, axis=1)

    o = jnp.dot(att, wo_ref[...], preferred_element_type=f32) + bo_ref[...]
    x1 = x + o

    mu2 = jnp.mean(x1, axis=-1, keepdims=True)
    var2 = jnp.mean(jnp.square(x1), axis=-1, keepdims=True) - jnp.square(mu2)
    hn = (x1 - mu2) * jax.lax.rsqrt(var2 + 1e-5) * ln2_g_ref[...] \
        + ln2_b_ref[...]
    mid = _swish(jnp.dot(hn, wm1_ref[...], preferred_element_type=f32)
                 + bm1_ref[...])
    y = x1 + jnp.dot(mid, wm2_ref[...], preferred_element_type=f32) \
        + bm2_ref[...]
    y_ref[0] = y


@functools.partial(jax.jit, static_argnums=())
def kernel(pairwise_g, coset_functions, mask, ln1_g, ln1_b,
           wn_w0, wn_b0, wn_w1, wn_b1, wn_w2, wn_b2,
           Wq, bq, Wk, bk, Wi, bi, Wo, bo,
           ln2_g, ln2_b, Wm1, bm1, Wm2, bm2):
    f32 = jnp.float32
    x = coset_functions
    wqkv = jnp.concatenate([Wq, Wk, Wi], axis=1)
    bqkv = jnp.concatenate([bq, bk, bi]).reshape(1, 3 * C)

    bf16 = jnp.bfloat16
    eye = jnp.eye(GRP, dtype=f32)
    w0p = jnp.kron(eye, wn_w0).astype(bf16)
    b0p = jnp.tile(wn_b0, GRP).reshape(1, KD * GRP)
    w1p = jnp.kron(eye, wn_w1).astype(bf16)
    b1p = jnp.tile(wn_b1, GRP).reshape(1, KD * GRP)
    G3 = 2 * GRP
    w2s = jnp.kron(jnp.eye(G3, dtype=f32), wn_w2)
    perm = [(t % G3) * H + t // G3 for t in range(H * G3)]
    w2p = w2s[:, jnp.array(perm)].astype(bf16)
    b2p = jnp.repeat(wn_b2, G3).reshape(1, H * G3)

    g2 = pairwise_g.reshape(B, N, N * G).astype(bf16)

    call = pl.pallas_call(
        _fused_kernel,
        grid=(B,),
        in_specs=[
            pl.BlockSpec((1, N, N * G), lambda b: (b, 0, 0)),
            pl.BlockSpec((1, N, C), lambda b: (b, 0, 0)),
            pl.BlockSpec((1, C), lambda b: (0, 0)),
            pl.BlockSpec((1, C), lambda b: (0, 0)),
            pl.BlockSpec((C, 3 * C), lambda b: (0, 0)),
            pl.BlockSpec((1, 3 * C), lambda b: (0, 0)),
            pl.BlockSpec((G * GRP, KD * GRP), lambda b: (0, 0)),
            pl.BlockSpec((1, KD * GRP), lambda b: (0, 0)),
            pl.BlockSpec((KD * GRP, KD * GRP), lambda b: (0, 0)),
            pl.BlockSpec((1, KD * GRP), lambda b: (0, 0)),
            pl.BlockSpec((KD * 2 * GRP, H * 2 * GRP), lambda b: (0, 0)),
            pl.BlockSpec((1, H * 2 * GRP), lambda b: (0, 0)),
            pl.BlockSpec((C, C), lambda b: (0, 0)),
            pl.BlockSpec((1, C), lambda b: (0, 0)),
            pl.BlockSpec((1, C), lambda b: (0, 0)),
            pl.BlockSpec((1, C), lambda b: (0, 0)),
            pl.BlockSpec((C, C), lambda b: (0, 0)),
            pl.BlockSpec((1, C), lambda b: (0, 0)),
            pl.BlockSpec((C, C), lambda b: (0, 0)),
            pl.BlockSpec((1, C), lambda b: (0, 0)),
        ],
        out_specs=pl.BlockSpec((1, N, C), lambda b: (b, 0, 0)),
        out_shape=jax.ShapeDtypeStruct((B, N, C), f32),
    )
    y = call(g2, x, ln1_g.reshape(1, C), ln1_b.reshape(1, C), wqkv, bqkv,
             w0p, b0p, w1p, b1p, w2p, b2p,
             Wo, bo.reshape(1, C), ln2_g.reshape(1, C), ln2_b.reshape(1, C),
             Wm1, bm1.reshape(1, C), Wm2, bm2.reshape(1, C))

    return (pairwise_g, y, mask)
